# baseline (speedup 1.0000x reference)
import jax
import jax.numpy as jnp
from jax.experimental import pallas as pl
from jax.experimental.pallas import tpu as pltpu

_P = 1024
_D = 1024
_TEMP = 0.01
_K = 103
_H = _D // 2


def _shear(U, base):
    A = U if base == 0 else jnp.roll(U, base, axis=0)
    col = jax.lax.broadcasted_iota(jnp.int32, (_P, _H), 1)
    for b in range(9):
        shift = 1 << b
        A = jnp.where((col & shift) != 0, jnp.roll(A, shift, axis=0), A)
    return A


def _body(x_hbm, V_hbm, alpha_ref, out_hbm,
          x_vmem, v0, v1, acc, sx, s0, s1, so):
    cpx = pltpu.make_async_copy(x_hbm, x_vmem, sx)
    cp0 = pltpu.make_async_copy(V_hbm.at[:, pl.ds(0, _H)], v0, s0)
    cp1 = pltpu.make_async_copy(V_hbm.at[:, pl.ds(_H, _H)], v1, s1)
    cpx.start()
    cp0.start()
    cp1.start()

    logits = alpha_ref[:, :] * (1.0 / _TEMP)
    m = jnp.max(logits, axis=0, keepdims=True)
    e = jnp.exp(logits - m)
    s = jnp.sum(e, axis=0, keepdims=True)
    aw = jnp.clip(e * (_K / s), 0.0, 1.0)

    cp0.wait()
    A0 = _shear((v0[:, :] * aw).astype(jnp.bfloat16), 0)
    cpx.wait()
    x16 = x_vmem[:, :].astype(jnp.bfloat16)
    acc[:, :] = jax.lax.dot_general(
        x16[:, 0:_H], A0, (((1,), (1,)), ((), ())),
        preferred_element_type=jnp.float32)

    cp1.wait()
    A1 = _shear((v1[:, :] * aw).astype(jnp.bfloat16), _H)
    acc[:, :] += jax.lax.dot_general(
        x16[:, _H:_D], A1, (((1,), (1,)), ((), ())),
        preferred_element_type=jnp.float32)

    cpo = pltpu.make_async_copy(acc, out_hbm, so)
    cpo.start()
    cpo.wait()


@jax.jit
def kernel(x, V, alpha):
    B = x.shape[0]
    return pl.pallas_call(
        _body,
        in_specs=[
            pl.BlockSpec(memory_space=pltpu.HBM),
            pl.BlockSpec(memory_space=pltpu.HBM),
            pl.BlockSpec((_P, 1), lambda: (0, 0)),
        ],
        out_specs=pl.BlockSpec(memory_space=pltpu.HBM),
        out_shape=jax.ShapeDtypeStruct((B, _P), x.dtype),
        scratch_shapes=[
            pltpu.VMEM((B, _D), jnp.float32),
            pltpu.VMEM((_P, _H), jnp.float32),
            pltpu.VMEM((_P, _H), jnp.float32),
            pltpu.VMEM((B, _P), jnp.float32),
            pltpu.SemaphoreType.DMA,
            pltpu.SemaphoreType.DMA,
            pltpu.SemaphoreType.DMA,
            pltpu.SemaphoreType.DMA,
        ],
    )(x, V, alpha.reshape(_P, 1))


# speedup vs baseline: 1.1554x; 1.1554x over previous
import jax
import jax.numpy as jnp
from jax.experimental import pallas as pl
from jax.experimental.pallas import tpu as pltpu

_P = 1024
_D = 1024
_TEMP = 0.01
_K = 103
_BLK = 256


def _body(x_ref, V_ref, alpha_ref, out_ref):
    logits = alpha_ref[:, :] * (1.0 / _TEMP)
    m = jnp.max(logits, axis=0, keepdims=True)
    e = jnp.exp(logits - m)
    s = jnp.sum(e, axis=0, keepdims=True)
    aw = jnp.clip(e * (_K / s), 0.0, 1.0)

    x16 = x_ref[:, :].astype(jnp.bfloat16)
    col = jax.lax.broadcasted_iota(jnp.int32, (_P, _BLK), 1)

    acc = None
    for k in range(_D // _BLK):
        blk = slice(k * _BLK, (k + 1) * _BLK)
        A = (V_ref[:, blk] * aw).astype(jnp.bfloat16)
        if k:
            A = jnp.roll(A, k * _BLK, axis=0)
        for b in range(8):
            shift = 1 << b
            A = jnp.where((col & shift) != 0, jnp.roll(A, shift, axis=0), A)
        part = jax.lax.dot_general(
            x16[:, blk], A, (((1,), (1,)), ((), ())),
            preferred_element_type=jnp.float32)
        acc = part if acc is None else acc + part

    out_ref[:, :] = acc


@jax.jit
def kernel(x, V, alpha):
    B = x.shape[0]
    return pl.pallas_call(
        _body,
        out_shape=jax.ShapeDtypeStruct((B, _P), x.dtype),
    )(x, V, alpha.reshape(_P, 1))
